# Initial kernel scaffold; baseline (speedup 1.0000x reference)
import jax
import jax.numpy as jnp
from jax.experimental import pallas as pl


def kernel(x, enc_w1, enc_b1, bn1_g, bn1_b, enc_w2, enc_b2, bn2_g, bn2_b, pre_w, pre_b, codebook, post_w, post_b, dec_w1, dec_b1, bn3_g, bn3_b, dec_w2, dec_b2):
    raise NotImplementedError("write your pallas kernel here")


# speedup vs baseline: 1.0435x; 1.0435x over previous
import jax
import jax.numpy as jnp
from jax.experimental import pallas as pl


def _conv2d(x, w, b, stride, pad):
    y = jax.lax.conv_general_dilated(
        x, w, (stride, stride), ((pad, pad), (pad, pad)),
        dimension_numbers=('NCHW', 'OIHW', 'NCHW'))
    return y + b[None, :, None, None]


def _conv_transpose2d(x, w, b, stride, pad):
    k = w.shape[2]
    w2 = jnp.transpose(w[:, :, ::-1, ::-1], (1, 0, 2, 3))
    B, C, H, W = x.shape
    xd = jnp.zeros((B, C, (H - 1) * stride + 1, (W - 1) * stride + 1), dtype=x.dtype)
    xd = xd.at[:, :, ::stride, ::stride].set(x)
    y = jax.lax.conv_general_dilated(
        xd, w2, (1, 1),
        ((k - 1 - pad, k - 1 - pad), (k - 1 - pad, k - 1 - pad)),
        dimension_numbers=('NCHW', 'OIHW', 'NCHW'))
    return y + b[None, :, None, None]


def _batch_norm(x, g, b, eps=1e-5):
    mean = jnp.mean(x, axis=(0, 2, 3), keepdims=True)
    var = jnp.var(x, axis=(0, 2, 3), keepdims=True)
    return (x - mean) / jnp.sqrt(var + eps) * g[None, :, None, None] + b[None, :, None, None]


def _latent_kernel(c2_ref, g_ref, b_ref, prew_ref, preb_ref, cb_ref,
                   postw_ref, postb_ref, z_ref, loss_ref):
    hs = []
    for c in range(4):
        ch = c2_ref[:, c, :, :]
        m = jnp.mean(ch)
        v = jnp.mean((ch - m) ** 2)
        inv = jax.lax.rsqrt(v + 1e-5)
        hn = (ch - m) * (inv * g_ref[0, c]) + b_ref[0, c]
        hs.append(jnp.maximum(hn, 0.0))
    q = []
    for o in range(2):
        s = hs[0] * prew_ref[o, 0]
        for c in range(1, 4):
            s = s + hs[c] * prew_ref[o, c]
        q.append(s + preb_ref[0, o])
    q0, q1 = q
    d0 = (q0 - cb_ref[0, 0]) ** 2 + (q1 - cb_ref[0, 1]) ** 2
    d1 = (q0 - cb_ref[1, 0]) ** 2 + (q1 - cb_ref[1, 1]) ** 2
    d2 = (q0 - cb_ref[2, 0]) ** 2 + (q1 - cb_ref[2, 1]) ** 2
    m01 = d0 <= d1
    dmin01 = jnp.where(m01, d0, d1)
    take2 = d2 < dmin01
    e0 = jnp.where(take2, cb_ref[2, 0], jnp.where(m01, cb_ref[0, 0], cb_ref[1, 0]))
    e1 = jnp.where(take2, cb_ref[2, 1], jnp.where(m01, cb_ref[0, 1], cb_ref[1, 1]))
    se = (e0 - q0) ** 2 + (e1 - q1) ** 2
    loss = 1.2 * 0.5 * jnp.mean(se)
    loss_ref[...] = jnp.full((8, 128), loss, dtype=jnp.float32)
    for o in range(4):
        z_ref[:, o, :, :] = (e0 * postw_ref[o, 0] + e1 * postw_ref[o, 1]
                             + postb_ref[0, o])


def kernel(x, enc_w1, enc_b1, bn1_g, bn1_b, enc_w2, enc_b2, bn2_g, bn2_b,
           pre_w, pre_b, codebook, post_w, post_b,
           dec_w1, dec_b1, bn3_g, bn3_b, dec_w2, dec_b2):
    h = jax.nn.relu(_batch_norm(_conv2d(x, enc_w1, enc_b1, 2, 1), bn1_g, bn1_b))
    c2 = _conv2d(h, enc_w2, enc_b2, 2, 1)
    z, lossbuf = pl.pallas_call(
        _latent_kernel,
        out_shape=[jax.ShapeDtypeStruct((16, 4, 128, 128), jnp.float32),
                   jax.ShapeDtypeStruct((8, 128), jnp.float32)],
    )(c2,
      bn2_g.reshape(1, 4), bn2_b.reshape(1, 4),
      pre_w.reshape(2, 4), pre_b.reshape(1, 2),
      codebook,
      post_w.reshape(4, 2), post_b.reshape(1, 4))
    quantize_loss = lossbuf[0, 0]
    d = jax.nn.relu(_batch_norm(_conv_transpose2d(z, dec_w1, dec_b1, 2, 1),
                                bn3_g, bn3_b))
    out = jnp.tanh(_conv_transpose2d(d, dec_w2, dec_b2, 2, 1))
    return (out, quantize_loss)
